# Initial kernel scaffold; baseline (speedup 1.0000x reference)
import jax
import jax.numpy as jnp
from jax.experimental import pallas as pl


def kernel(x, edge_index, W1, b1, W3, b3, W2, b2):
    raise NotImplementedError("write your pallas kernel here")


# speedup vs baseline: 25.7605x; 25.7605x over previous
import functools

import jax
import jax.numpy as jnp
from jax import lax
from jax.experimental import pallas as pl
from jax.experimental.pallas import tpu as pltpu
from jax.experimental.pallas import tpu_sc as plsc

N = 10000
NP = 10240
D = 16
E = 320000
NSC = 2
NTILE = 16
NW = NSC * NTILE
CHUNK = 128
NCHUNK = 79
EPW = CHUNK * NCHUNK
EP = EPW * NW
ROWS_PER_TILE = NP // NTILE

_MESH = plsc.VectorSubcoreMesh(core_axis_name="c", subcore_axis_name="s")
_SC_PARAMS = pltpu.CompilerParams(use_tc_tiling_on_sc=False)


def _sc_deg_body(col_hbm, ones_hbm, zero_hbm, out_hbm, col_v, ones_v, accum_s):
    c = lax.axis_index("c")
    s = lax.axis_index("s")
    wid = c * NTILE + s
    rbase = s * ROWS_PER_TILE

    pltpu.sync_copy(zero_hbm.at[pl.ds(rbase, ROWS_PER_TILE)],
                    accum_s.at[pl.ds(rbase, ROWS_PER_TILE)])
    pltpu.sync_copy(col_hbm.at[wid], col_v)
    pltpu.sync_copy(ones_hbm, ones_v)
    plsc.subcore_barrier()

    def body(j, carry):
        pltpu.sync_copy(ones_v, accum_s.at[col_v.at[j]], add=True)
        return carry

    lax.fori_loop(0, NCHUNK, body, 0)

    plsc.subcore_barrier()
    pltpu.sync_copy(accum_s.at[pl.ds(rbase, ROWS_PER_TILE)],
                    out_hbm.at[c, pl.ds(rbase, ROWS_PER_TILE)])


_sc_deg = functools.partial(
    pl.kernel,
    out_type=jax.ShapeDtypeStruct((NSC, NP, D), jnp.float32),
    mesh=_MESH,
    scratch_types=[
        pltpu.VMEM((NCHUNK, CHUNK), jnp.int32),
        pltpu.VMEM((CHUNK, D), jnp.float32),
        pltpu.VMEM_SHARED((NP, D), jnp.float32),
    ],
    compiler_params=_SC_PARAMS,
)(_sc_deg_body)


def _sc_agg_body(g_hbm, row_hbm, col_hbm, zero_hbm, out_hbm,
                 row_v, col_v, data_v, accum_s):
    c = lax.axis_index("c")
    s = lax.axis_index("s")
    wid = c * NTILE + s
    rbase = s * ROWS_PER_TILE

    @pl.when(c == 0)
    def _():
        pltpu.sync_copy(g_hbm.at[pl.ds(rbase, ROWS_PER_TILE)],
                        accum_s.at[pl.ds(rbase, ROWS_PER_TILE)])

    @pl.when(c != 0)
    def _():
        pltpu.sync_copy(zero_hbm.at[pl.ds(rbase, ROWS_PER_TILE)],
                        accum_s.at[pl.ds(rbase, ROWS_PER_TILE)])

    pltpu.sync_copy(row_hbm.at[wid], row_v)
    pltpu.sync_copy(col_hbm.at[wid], col_v)
    plsc.subcore_barrier()

    def body(j, carry):
        pltpu.sync_copy(g_hbm.at[row_v.at[j]], data_v)
        pltpu.sync_copy(data_v, accum_s.at[col_v.at[j]], add=True)
        return carry

    lax.fori_loop(0, NCHUNK, body, 0)

    plsc.subcore_barrier()
    pltpu.sync_copy(accum_s.at[pl.ds(rbase, ROWS_PER_TILE)],
                    out_hbm.at[c, pl.ds(rbase, ROWS_PER_TILE)])


_sc_agg = functools.partial(
    pl.kernel,
    out_type=jax.ShapeDtypeStruct((NSC, NP, D), jnp.float32),
    mesh=_MESH,
    scratch_types=[
        pltpu.VMEM((NCHUNK, CHUNK), jnp.int32),
        pltpu.VMEM((NCHUNK, CHUNK), jnp.int32),
        pltpu.VMEM((CHUNK, D), jnp.float32),
        pltpu.VMEM_SHARED((NP, D), jnp.float32),
    ],
    compiler_params=_SC_PARAMS,
)(_sc_agg_body)


BR = 1280
GRID = NP // BR

_row_spec = pl.BlockSpec((BR, D), lambda i: (i, 0))


def _tc1_body(x_ref, w_ref, da_ref, db_ref, g_ref, dinv_ref):
    deg = da_ref[...] + db_ref[...] + 1.0
    dinv = lax.rsqrt(deg)
    h = jnp.dot(x_ref[...], w_ref[...], preferred_element_type=jnp.float32)
    dinv_ref[...] = dinv
    g_ref[...] = h * dinv


def _tc1(x_p, w1, deg_a, deg_b):
    return pl.pallas_call(
        _tc1_body,
        grid=(GRID,),
        in_specs=[
            pl.BlockSpec((BR, 128), lambda i: (i, 0)),
            pl.BlockSpec((128, D), lambda i: (0, 0)),
            _row_spec,
            _row_spec,
        ],
        out_specs=[_row_spec, _row_spec],
        out_shape=[jax.ShapeDtypeStruct((NP, D), jnp.float32),
                   jax.ShapeDtypeStruct((NP, D), jnp.float32)],
    )(x_p, w1, deg_a, deg_b)


def _tc_mid_body(aa_ref, ab_ref, dinv_ref, b_ref, w_ref, g_ref):
    dinv = dinv_ref[...]
    z = (aa_ref[...] + ab_ref[...]) * dinv + b_ref[0:1, :]
    h = jnp.maximum(z, 0.0)
    g_ref[...] = jnp.dot(h, w_ref[...], preferred_element_type=jnp.float32) * dinv


def _tc_mid(acc_a, acc_b, dinv, b_tiled, w):
    return pl.pallas_call(
        _tc_mid_body,
        grid=(GRID,),
        in_specs=[
            _row_spec,
            _row_spec,
            _row_spec,
            pl.BlockSpec((8, D), lambda i: (0, 0)),
            pl.BlockSpec((D, D), lambda i: (0, 0)),
        ],
        out_specs=_row_spec,
        out_shape=jax.ShapeDtypeStruct((NP, D), jnp.float32),
    )(acc_a, acc_b, dinv, b_tiled, w)


def _tc_fin_body(aa_ref, ab_ref, dinv_ref, w_ref, b_ref, o_ref):
    r = (aa_ref[...] + ab_ref[...]) * dinv_ref[...]
    z = jnp.dot(r, w_ref[...], preferred_element_type=jnp.float32) + b_ref[0:1, :]
    m = jnp.max(z, axis=1, keepdims=True)
    zs = z - m
    o_ref[...] = zs - jnp.log(jnp.sum(jnp.exp(zs), axis=1, keepdims=True))


def _tc_fin(acc_a, acc_b, dinv, w2, b2_tiled):
    return pl.pallas_call(
        _tc_fin_body,
        grid=(GRID,),
        in_specs=[
            _row_spec,
            _row_spec,
            _row_spec,
            pl.BlockSpec((D, 2), lambda i: (0, 0)),
            pl.BlockSpec((8, 2), lambda i: (0, 0)),
        ],
        out_specs=pl.BlockSpec((BR, 2), lambda i: (i, 0)),
        out_shape=jax.ShapeDtypeStruct((NP, 2), jnp.float32),
    )(acc_a, acc_b, dinv, w2, b2_tiled)


def kernel(x, edge_index, W1, b1, W3, b3, W2, b2):
    row = edge_index[0].astype(jnp.int32)
    col = edge_index[1].astype(jnp.int32)
    pad = EP - E
    row3 = jnp.concatenate(
        [row, jnp.zeros((pad,), jnp.int32)]).reshape(NW, NCHUNK, CHUNK)
    col3 = jnp.concatenate(
        [col, jnp.full((pad,), NP - 1, jnp.int32)]).reshape(NW, NCHUNK, CHUNK)

    x_p = jnp.pad(x, ((0, NP - N), (0, 0)))
    zeros = jnp.zeros((NP, D), jnp.float32)
    ones_chunk = jnp.ones((CHUNK, D), jnp.float32)
    b1_t = jnp.tile(b1.reshape(1, D), (8, 1))
    b3_t = jnp.tile(b3.reshape(1, D), (8, 1))
    b2_t = jnp.tile(b2.reshape(1, 2), (8, 1))
    eye = jnp.eye(D, dtype=jnp.float32)

    deg2 = _sc_deg(col3, ones_chunk, zeros)
    g1, dinv = _tc1(x_p, W1, deg2[0], deg2[1])
    a1 = _sc_agg(g1, row3, col3, zeros)
    g2 = _tc_mid(a1[0], a1[1], dinv, b1_t, W3)
    a2 = _sc_agg(g2, row3, col3, zeros)
    g3 = _tc_mid(a2[0], a2[1], dinv, b3_t, eye)
    a3 = _sc_agg(g3, row3, col3, zeros)
    out = _tc_fin(a3[0], a3[1], dinv, W2, b2_t)
    return out[:N]


# speedup vs baseline: 33.2616x; 1.2912x over previous
import functools

import jax
import jax.numpy as jnp
from jax import lax
from jax.experimental import pallas as pl
from jax.experimental.pallas import tpu as pltpu
from jax.experimental.pallas import tpu_sc as plsc

N = 10000
NP = 10240
D = 16
E = 320000
NSC = 2
NTILE = 16
NW = NSC * NTILE
CHUNK = 128
NCHUNK = 80
NBUF = 4
NGRP = NCHUNK // NBUF
EPW = CHUNK * NCHUNK
EP = EPW * NW
ROWS_PER_TILE = NP // NTILE

_MESH = plsc.VectorSubcoreMesh(core_axis_name="c", subcore_axis_name="s")
_SC_PARAMS = pltpu.CompilerParams(use_tc_tiling_on_sc=False)


def _sc_deg_body(col_hbm, ones_hbm, zero_hbm, out_hbm, col_v, ones_v, accum_s):
    c = lax.axis_index("c")
    s = lax.axis_index("s")
    wid = c * NTILE + s
    rbase = s * ROWS_PER_TILE

    pltpu.sync_copy(zero_hbm.at[pl.ds(rbase, ROWS_PER_TILE)],
                    accum_s.at[pl.ds(rbase, ROWS_PER_TILE)])
    pltpu.sync_copy(col_hbm.at[wid], col_v)
    pltpu.sync_copy(ones_hbm, ones_v)
    plsc.subcore_barrier()

    def body(j, carry):
        pltpu.sync_copy(ones_v, accum_s.at[col_v.at[j]], add=True)
        return carry

    lax.fori_loop(0, NCHUNK, body, 0)

    plsc.subcore_barrier()
    pltpu.sync_copy(accum_s.at[pl.ds(rbase, ROWS_PER_TILE)],
                    out_hbm.at[c, pl.ds(rbase, ROWS_PER_TILE)])


_sc_deg = functools.partial(
    pl.kernel,
    out_type=jax.ShapeDtypeStruct((NSC, NP, D), jnp.float32),
    mesh=_MESH,
    scratch_types=[
        pltpu.VMEM((NCHUNK, CHUNK), jnp.int32),
        pltpu.VMEM((CHUNK, D), jnp.float32),
        pltpu.VMEM_SHARED((NP, D), jnp.float32),
    ],
    compiler_params=_SC_PARAMS,
)(_sc_deg_body)


def _sc_agg_body(g_hbm, row_hbm, col_hbm, zero_hbm, out_hbm,
                 row_v, col_v, data_v, accum_s, s0, s1, s2, s3):
    c = lax.axis_index("c")
    s = lax.axis_index("s")
    wid = c * NTILE + s
    rbase = s * ROWS_PER_TILE
    sems = (s0, s1, s2, s3)

    @pl.when(c == 0)
    def _():
        pltpu.sync_copy(g_hbm.at[pl.ds(rbase, ROWS_PER_TILE)],
                        accum_s.at[pl.ds(rbase, ROWS_PER_TILE)])

    @pl.when(c != 0)
    def _():
        pltpu.sync_copy(zero_hbm.at[pl.ds(rbase, ROWS_PER_TILE)],
                        accum_s.at[pl.ds(rbase, ROWS_PER_TILE)])

    pltpu.sync_copy(row_hbm.at[wid], row_v)
    pltpu.sync_copy(col_hbm.at[wid], col_v)
    plsc.subcore_barrier()

    for b in range(NBUF):
        pltpu.async_copy(g_hbm.at[row_v.at[b]], data_v.at[b], sems[b])

    def group(gi, carry):
        for b in range(NBUF):
            j = gi * NBUF + b
            pltpu.make_async_copy(g_hbm.at[row_v.at[j]],
                                  data_v.at[b], sems[b]).wait()
            pltpu.sync_copy(data_v.at[b], accum_s.at[col_v.at[j]], add=True)

            @pl.when(j + NBUF < NCHUNK)
            def _():
                pltpu.async_copy(g_hbm.at[row_v.at[j + NBUF]],
                                 data_v.at[b], sems[b])
        return carry

    lax.fori_loop(0, NGRP, group, 0)

    plsc.subcore_barrier()
    pltpu.sync_copy(accum_s.at[pl.ds(rbase, ROWS_PER_TILE)],
                    out_hbm.at[c, pl.ds(rbase, ROWS_PER_TILE)])


_sc_agg = functools.partial(
    pl.kernel,
    out_type=jax.ShapeDtypeStruct((NSC, NP, D), jnp.float32),
    mesh=_MESH,
    scratch_types=[
        pltpu.VMEM((NCHUNK, CHUNK), jnp.int32),
        pltpu.VMEM((NCHUNK, CHUNK), jnp.int32),
        pltpu.VMEM((NBUF, CHUNK, D), jnp.float32),
        pltpu.VMEM_SHARED((NP, D), jnp.float32),
        pltpu.SemaphoreType.DMA,
        pltpu.SemaphoreType.DMA,
        pltpu.SemaphoreType.DMA,
        pltpu.SemaphoreType.DMA,
    ],
    compiler_params=_SC_PARAMS,
)(_sc_agg_body)


BR = 1280
GRID = NP // BR

_row_spec = pl.BlockSpec((BR, D), lambda i: (i, 0))


def _tc1_body(x_ref, w_ref, da_ref, db_ref, g_ref, dinv_ref):
    deg = da_ref[...] + db_ref[...] + 1.0
    dinv = lax.rsqrt(deg)
    h = jnp.dot(x_ref[...], w_ref[...], preferred_element_type=jnp.float32)
    dinv_ref[...] = dinv
    g_ref[...] = h * dinv


def _tc1(x_p, w1, deg_a, deg_b):
    return pl.pallas_call(
        _tc1_body,
        grid=(GRID,),
        in_specs=[
            pl.BlockSpec((BR, 128), lambda i: (i, 0)),
            pl.BlockSpec((128, D), lambda i: (0, 0)),
            _row_spec,
            _row_spec,
        ],
        out_specs=[_row_spec, _row_spec],
        out_shape=[jax.ShapeDtypeStruct((NP, D), jnp.float32),
                   jax.ShapeDtypeStruct((NP, D), jnp.float32)],
    )(x_p, w1, deg_a, deg_b)


def _tc_mid_body(aa_ref, ab_ref, dinv_ref, b_ref, w_ref, g_ref):
    dinv = dinv_ref[...]
    z = (aa_ref[...] + ab_ref[...]) * dinv + b_ref[0:1, :]
    h = jnp.maximum(z, 0.0)
    g_ref[...] = jnp.dot(h, w_ref[...], preferred_element_type=jnp.float32) * dinv


def _tc_mid(acc_a, acc_b, dinv, b_tiled, w):
    return pl.pallas_call(
        _tc_mid_body,
        grid=(GRID,),
        in_specs=[
            _row_spec,
            _row_spec,
            _row_spec,
            pl.BlockSpec((8, D), lambda i: (0, 0)),
            pl.BlockSpec((D, D), lambda i: (0, 0)),
        ],
        out_specs=_row_spec,
        out_shape=jax.ShapeDtypeStruct((NP, D), jnp.float32),
    )(acc_a, acc_b, dinv, b_tiled, w)


def _tc_fin_body(aa_ref, ab_ref, dinv_ref, w_ref, b_ref, o_ref):
    r = (aa_ref[...] + ab_ref[...]) * dinv_ref[...]
    z = jnp.dot(r, w_ref[...], preferred_element_type=jnp.float32) + b_ref[0:1, :]
    m = jnp.max(z, axis=1, keepdims=True)
    zs = z - m
    o_ref[...] = zs - jnp.log(jnp.sum(jnp.exp(zs), axis=1, keepdims=True))


def _tc_fin(acc_a, acc_b, dinv, w2, b2_tiled):
    return pl.pallas_call(
        _tc_fin_body,
        grid=(GRID,),
        in_specs=[
            _row_spec,
            _row_spec,
            _row_spec,
            pl.BlockSpec((D, 2), lambda i: (0, 0)),
            pl.BlockSpec((8, 2), lambda i: (0, 0)),
        ],
        out_specs=pl.BlockSpec((BR, 2), lambda i: (i, 0)),
        out_shape=jax.ShapeDtypeStruct((NP, 2), jnp.float32),
    )(acc_a, acc_b, dinv, w2, b2_tiled)


def kernel(x, edge_index, W1, b1, W3, b3, W2, b2):
    row = edge_index[0].astype(jnp.int32)
    col = edge_index[1].astype(jnp.int32)
    pad = EP - E
    row3 = jnp.concatenate(
        [row, jnp.zeros((pad,), jnp.int32)]).reshape(NW, NCHUNK, CHUNK)
    col3 = jnp.concatenate(
        [col, jnp.full((pad,), NP - 1, jnp.int32)]).reshape(NW, NCHUNK, CHUNK)

    x_p = jnp.pad(x, ((0, NP - N), (0, 0)))
    zeros = jnp.zeros((NP, D), jnp.float32)
    ones_chunk = jnp.ones((CHUNK, D), jnp.float32)
    b1_t = jnp.tile(b1.reshape(1, D), (8, 1))
    b3_t = jnp.tile(b3.reshape(1, D), (8, 1))
    b2_t = jnp.tile(b2.reshape(1, 2), (8, 1))
    eye = jnp.eye(D, dtype=jnp.float32)

    deg2 = _sc_deg(col3, ones_chunk, zeros)
    g1, dinv = _tc1(x_p, W1, deg2[0], deg2[1])
    a1 = _sc_agg(g1, row3, col3, zeros)
    g2 = _tc_mid(a1[0], a1[1], dinv, b1_t, W3)
    a2 = _sc_agg(g2, row3, col3, zeros)
    g3 = _tc_mid(a2[0], a2[1], dinv, b3_t, eye)
    a3 = _sc_agg(g3, row3, col3, zeros)
    out = _tc_fin(a3[0], a3[1], dinv, W2, b2_t)
    return out[:N]


# speedup vs baseline: 45.5637x; 1.3699x over previous
import functools

import jax
import jax.numpy as jnp
from jax import lax
from jax.experimental import pallas as pl
from jax.experimental.pallas import tpu as pltpu
from jax.experimental.pallas import tpu_sc as plsc

N = 10000
NP = 10240
D = 16
E = 320000
NSC = 2
NTILE = 16
NW = NSC * NTILE
CHUNK = 128
NCHUNK = 80
NBUF = 4
NGRP = NCHUNK // NBUF
EPW = CHUNK * NCHUNK
EP = EPW * NW
ROWS_PER_TILE = NP // NTILE

_MESH = plsc.VectorSubcoreMesh(core_axis_name="c", subcore_axis_name="s")
_SC_PARAMS = pltpu.CompilerParams(use_tc_tiling_on_sc=False)


def _sc_deg_body(col_hbm, ones_hbm, zero_hbm, out_hbm, col_v, ones_v, accum_s):
    c = lax.axis_index("c")
    s = lax.axis_index("s")
    wid = c * NTILE + s
    rbase = s * ROWS_PER_TILE

    pltpu.sync_copy(zero_hbm.at[pl.ds(rbase, ROWS_PER_TILE)],
                    accum_s.at[pl.ds(rbase, ROWS_PER_TILE)])
    pltpu.sync_copy(col_hbm.at[wid], col_v)
    pltpu.sync_copy(ones_hbm, ones_v)
    plsc.subcore_barrier()

    def body(j, carry):
        pltpu.sync_copy(ones_v, accum_s.at[col_v.at[j]], add=True)
        return carry

    lax.fori_loop(0, NCHUNK, body, 0)

    plsc.subcore_barrier()
    pltpu.sync_copy(accum_s.at[pl.ds(rbase, ROWS_PER_TILE)],
                    out_hbm.at[c, pl.ds(rbase, ROWS_PER_TILE)])


_sc_deg = functools.partial(
    pl.kernel,
    out_type=jax.ShapeDtypeStruct((NSC, NP, D), jnp.float32),
    mesh=_MESH,
    scratch_types=[
        pltpu.VMEM((NCHUNK, CHUNK), jnp.int32),
        pltpu.VMEM((CHUNK, D), jnp.float32),
        pltpu.VMEM_SHARED((NP, D), jnp.float32),
    ],
    compiler_params=_SC_PARAMS,
)(_sc_deg_body)


def _sc_agg_body(g_hbm, row_hbm, col_hbm, zero_hbm, out_hbm,
                 row_v, col_v, data_v, accum_s, s0, s1, s2, s3):
    c = lax.axis_index("c")
    s = lax.axis_index("s")
    wid = c * NTILE + s
    rbase = s * ROWS_PER_TILE
    sems = (s0, s1, s2, s3)

    @pl.when(c == 0)
    def _():
        pltpu.sync_copy(g_hbm.at[pl.ds(rbase, ROWS_PER_TILE)],
                        accum_s.at[pl.ds(rbase, ROWS_PER_TILE)])

    @pl.when(c != 0)
    def _():
        pltpu.sync_copy(zero_hbm.at[pl.ds(rbase, ROWS_PER_TILE)],
                        accum_s.at[pl.ds(rbase, ROWS_PER_TILE)])

    pltpu.sync_copy(row_hbm.at[wid], row_v)
    pltpu.sync_copy(col_hbm.at[wid], col_v)
    plsc.subcore_barrier()

    for b in range(NBUF):
        pltpu.async_copy(g_hbm.at[row_v.at[b]], data_v.at[b], sems[b])

    def group(gi, carry):
        for b in range(NBUF):
            j = gi * NBUF + b
            pltpu.make_async_copy(g_hbm.at[row_v.at[j]],
                                  data_v.at[b], sems[b]).wait()
            pltpu.sync_copy(data_v.at[b], accum_s.at[col_v.at[j]], add=True)

            @pl.when(j + NBUF < NCHUNK)
            def _():
                pltpu.async_copy(g_hbm.at[row_v.at[j + NBUF]],
                                 data_v.at[b], sems[b])
        return carry

    lax.fori_loop(0, NGRP, group, 0)

    plsc.subcore_barrier()
    pltpu.sync_copy(accum_s.at[pl.ds(rbase, ROWS_PER_TILE)],
                    out_hbm.at[c, pl.ds(rbase, ROWS_PER_TILE)])


_sc_agg = functools.partial(
    pl.kernel,
    out_type=jax.ShapeDtypeStruct((NSC, NP, D), jnp.float32),
    mesh=_MESH,
    scratch_types=[
        pltpu.VMEM((NCHUNK, CHUNK), jnp.int32),
        pltpu.VMEM((NCHUNK, CHUNK), jnp.int32),
        pltpu.VMEM((NBUF, CHUNK, D), jnp.float32),
        pltpu.VMEM_SHARED((NP, D), jnp.float32),
        pltpu.SemaphoreType.DMA,
        pltpu.SemaphoreType.DMA,
        pltpu.SemaphoreType.DMA,
        pltpu.SemaphoreType.DMA,
    ],
    compiler_params=_SC_PARAMS,
)(_sc_agg_body)


BR = 1280
GRID = NP // BR

_row_spec = pl.BlockSpec((BR, D), lambda i: (i, 0))


def _tc1_body(x_ref, w_ref, da_ref, db_ref, g_ref, dinv_ref):
    deg = da_ref[...] + db_ref[...] + 1.0
    dinv = lax.rsqrt(deg)
    h = jnp.dot(x_ref[...], w_ref[...], preferred_element_type=jnp.float32)
    dinv_ref[...] = dinv
    g_ref[...] = h * dinv


def _tc1(x_p, w1, deg_a, deg_b):
    return pl.pallas_call(
        _tc1_body,
        grid=(GRID,),
        in_specs=[
            pl.BlockSpec((BR, 128), lambda i: (i, 0)),
            pl.BlockSpec((128, D), lambda i: (0, 0)),
            _row_spec,
            _row_spec,
        ],
        out_specs=[_row_spec, _row_spec],
        out_shape=[jax.ShapeDtypeStruct((NP, D), jnp.float32),
                   jax.ShapeDtypeStruct((NP, D), jnp.float32)],
    )(x_p, w1, deg_a, deg_b)


def _tc_mid_body(aa_ref, ab_ref, dinv_ref, b_ref, w_ref, g_ref):
    dinv = dinv_ref[...]
    z = (aa_ref[...] + ab_ref[...]) * dinv + b_ref[0:1, :]
    h = jnp.maximum(z, 0.0)
    g_ref[...] = jnp.dot(h, w_ref[...], preferred_element_type=jnp.float32) * dinv


def _tc_mid(acc_a, acc_b, dinv, b_tiled, w):
    return pl.pallas_call(
        _tc_mid_body,
        grid=(GRID,),
        in_specs=[
            _row_spec,
            _row_spec,
            _row_spec,
            pl.BlockSpec((8, D), lambda i: (0, 0)),
            pl.BlockSpec((D, D), lambda i: (0, 0)),
        ],
        out_specs=_row_spec,
        out_shape=jax.ShapeDtypeStruct((NP, D), jnp.float32),
    )(acc_a, acc_b, dinv, b_tiled, w)


def _tc_fin_body(aa_ref, ab_ref, dinv_ref, w_ref, b_ref, o_ref):
    r = (aa_ref[...] + ab_ref[...]) * dinv_ref[...]
    z = jnp.dot(r, w_ref[...], preferred_element_type=jnp.float32) + b_ref[0:1, :]
    m = jnp.max(z, axis=1, keepdims=True)
    zs = z - m
    o_ref[...] = zs - jnp.log(jnp.sum(jnp.exp(zs), axis=1, keepdims=True))


def _tc_fin(acc_a, acc_b, dinv, w2, b2_tiled):
    return pl.pallas_call(
        _tc_fin_body,
        grid=(GRID,),
        in_specs=[
            _row_spec,
            _row_spec,
            _row_spec,
            pl.BlockSpec((D, 2), lambda i: (0, 0)),
            pl.BlockSpec((8, 2), lambda i: (0, 0)),
        ],
        out_specs=pl.BlockSpec((BR, 2), lambda i: (i, 0)),
        out_shape=jax.ShapeDtypeStruct((NP, 2), jnp.float32),
    )(acc_a, acc_b, dinv, w2, b2_tiled)


def kernel(x, edge_index, W1, b1, W3, b3, W2, b2):
    row = edge_index[0].astype(jnp.int32)
    col = edge_index[1].astype(jnp.int32)
    pad = EP - E
    pad_idx = jnp.arange(pad, dtype=jnp.int32)
    row3 = jnp.concatenate(
        [row, pad_idx % N]).reshape(NW, NCHUNK, CHUNK)
    col3 = jnp.concatenate(
        [col, N + pad_idx % (NP - N)]).reshape(NW, NCHUNK, CHUNK)

    x_p = jnp.pad(x, ((0, NP - N), (0, 0)))
    zeros = jnp.zeros((NP, D), jnp.float32)
    ones_chunk = jnp.ones((CHUNK, D), jnp.float32)
    b1_t = jnp.tile(b1.reshape(1, D), (8, 1))
    b3_t = jnp.tile(b3.reshape(1, D), (8, 1))
    b2_t = jnp.tile(b2.reshape(1, 2), (8, 1))
    eye = jnp.eye(D, dtype=jnp.float32)

    deg2 = _sc_deg(col3, ones_chunk, zeros)
    g1, dinv = _tc1(x_p, W1, deg2[0], deg2[1])
    a1 = _sc_agg(g1, row3, col3, zeros)
    g2 = _tc_mid(a1[0], a1[1], dinv, b1_t, W3)
    a2 = _sc_agg(g2, row3, col3, zeros)
    g3 = _tc_mid(a2[0], a2[1], dinv, b3_t, eye)
    a3 = _sc_agg(g3, row3, col3, zeros)
    out = _tc_fin(a3[0], a3[1], dinv, W2, b2_t)
    return out[:N]


# speedup vs baseline: 48.1628x; 1.0570x over previous
import functools

import jax
import jax.numpy as jnp
from jax import lax
from jax.experimental import pallas as pl
from jax.experimental.pallas import tpu as pltpu
from jax.experimental.pallas import tpu_sc as plsc

N = 10000
NP = 10240
D = 16
E = 320000
NSC = 2
NTILE = 16
NW = NSC * NTILE
CHUNK = 128
NCHUNK = 80
NBUF = 8
NGRP = NCHUNK // NBUF
EPW = CHUNK * NCHUNK
EP = EPW * NW
ROWS_PER_TILE = NP // NTILE

_MESH = plsc.VectorSubcoreMesh(core_axis_name="c", subcore_axis_name="s")
_SC_PARAMS = pltpu.CompilerParams(use_tc_tiling_on_sc=False)


def _sc_deg_body(col_hbm, ones_hbm, zero_hbm, out_hbm, col_v, ones_v, accum_s):
    c = lax.axis_index("c")
    s = lax.axis_index("s")
    wid = c * NTILE + s
    rbase = s * ROWS_PER_TILE

    pltpu.sync_copy(zero_hbm.at[pl.ds(rbase, ROWS_PER_TILE)],
                    accum_s.at[pl.ds(rbase, ROWS_PER_TILE)])
    pltpu.sync_copy(col_hbm.at[wid], col_v)
    pltpu.sync_copy(ones_hbm, ones_v)
    plsc.subcore_barrier()

    def body(j, carry):
        pltpu.sync_copy(ones_v, accum_s.at[col_v.at[j]], add=True)
        return carry

    lax.fori_loop(0, NCHUNK, body, 0)

    plsc.subcore_barrier()
    pltpu.sync_copy(accum_s.at[pl.ds(rbase, ROWS_PER_TILE)],
                    out_hbm.at[c, pl.ds(rbase, ROWS_PER_TILE)])


_sc_deg = functools.partial(
    pl.kernel,
    out_type=jax.ShapeDtypeStruct((NSC, NP, D), jnp.float32),
    mesh=_MESH,
    scratch_types=[
        pltpu.VMEM((NCHUNK, CHUNK), jnp.int32),
        pltpu.VMEM((CHUNK, D), jnp.float32),
        pltpu.VMEM_SHARED((NP, D), jnp.float32),
    ],
    compiler_params=_SC_PARAMS,
)(_sc_deg_body)


def _sc_agg_body(g_hbm, row_hbm, col_hbm, zero_hbm, out_hbm,
                 row_v, col_v, data_v, accum_s,
                 g0, g1, g2, g3, g4, g5, g6, g7,
                 t0, t1, t2, t3, t4, t5, t6, t7):
    c = lax.axis_index("c")
    s = lax.axis_index("s")
    wid = c * NTILE + s
    rbase = s * ROWS_PER_TILE
    gsems = (g0, g1, g2, g3, g4, g5, g6, g7)
    ssems = (t0, t1, t2, t3, t4, t5, t6, t7)

    @pl.when(c == 0)
    def _():
        pltpu.sync_copy(g_hbm.at[pl.ds(rbase, ROWS_PER_TILE)],
                        accum_s.at[pl.ds(rbase, ROWS_PER_TILE)])

    @pl.when(c != 0)
    def _():
        pltpu.sync_copy(zero_hbm.at[pl.ds(rbase, ROWS_PER_TILE)],
                        accum_s.at[pl.ds(rbase, ROWS_PER_TILE)])

    pltpu.sync_copy(row_hbm.at[wid], row_v)
    pltpu.sync_copy(col_hbm.at[wid], col_v)
    plsc.subcore_barrier()

    for b in range(NBUF):
        pltpu.async_copy(g_hbm.at[row_v.at[b]], data_v.at[b], gsems[b])

    def group(gi, carry):
        for b in range(NBUF):
            j = gi * NBUF + b
            pltpu.make_async_copy(g_hbm.at[row_v.at[j]],
                                  data_v.at[b], gsems[b]).wait()
            pltpu.async_copy(data_v.at[b], accum_s.at[col_v.at[j]],
                             ssems[b], add=True)
        for b in range(NBUF):
            j = gi * NBUF + b
            pltpu.make_async_copy(data_v.at[b], accum_s.at[col_v.at[j]],
                                  ssems[b]).wait()

            @pl.when(j + NBUF < NCHUNK)
            def _():
                pltpu.async_copy(g_hbm.at[row_v.at[j + NBUF]],
                                 data_v.at[b], gsems[b])
        return carry

    lax.fori_loop(0, NGRP, group, 0)

    plsc.subcore_barrier()
    pltpu.sync_copy(accum_s.at[pl.ds(rbase, ROWS_PER_TILE)],
                    out_hbm.at[c, pl.ds(rbase, ROWS_PER_TILE)])


_sc_agg = functools.partial(
    pl.kernel,
    out_type=jax.ShapeDtypeStruct((NSC, NP, D), jnp.float32),
    mesh=_MESH,
    scratch_types=[
        pltpu.VMEM((NCHUNK, CHUNK), jnp.int32),
        pltpu.VMEM((NCHUNK, CHUNK), jnp.int32),
        pltpu.VMEM((NBUF, CHUNK, D), jnp.float32),
        pltpu.VMEM_SHARED((NP, D), jnp.float32),
    ] + [pltpu.SemaphoreType.DMA] * (2 * NBUF),
    compiler_params=_SC_PARAMS,
)(_sc_agg_body)


BR = 1280
GRID = NP // BR

_row_spec = pl.BlockSpec((BR, D), lambda i: (i, 0))


def _tc1_body(x_ref, w_ref, da_ref, db_ref, g_ref, dinv_ref):
    deg = da_ref[...] + db_ref[...] + 1.0
    dinv = lax.rsqrt(deg)
    h = jnp.dot(x_ref[...], w_ref[...], preferred_element_type=jnp.float32)
    dinv_ref[...] = dinv
    g_ref[...] = h * dinv


def _tc1(x_p, w1, deg_a, deg_b):
    return pl.pallas_call(
        _tc1_body,
        grid=(GRID,),
        in_specs=[
            pl.BlockSpec((BR, 128), lambda i: (i, 0)),
            pl.BlockSpec((128, D), lambda i: (0, 0)),
            _row_spec,
            _row_spec,
        ],
        out_specs=[_row_spec, _row_spec],
        out_shape=[jax.ShapeDtypeStruct((NP, D), jnp.float32),
                   jax.ShapeDtypeStruct((NP, D), jnp.float32)],
    )(x_p, w1, deg_a, deg_b)


def _tc_mid_body(aa_ref, ab_ref, dinv_ref, b_ref, w_ref, g_ref):
    dinv = dinv_ref[...]
    z = (aa_ref[...] + ab_ref[...]) * dinv + b_ref[0:1, :]
    h = jnp.maximum(z, 0.0)
    g_ref[...] = jnp.dot(h, w_ref[...], preferred_element_type=jnp.float32) * dinv


def _tc_mid(acc_a, acc_b, dinv, b_tiled, w):
    return pl.pallas_call(
        _tc_mid_body,
        grid=(GRID,),
        in_specs=[
            _row_spec,
            _row_spec,
            _row_spec,
            pl.BlockSpec((8, D), lambda i: (0, 0)),
            pl.BlockSpec((D, D), lambda i: (0, 0)),
        ],
        out_specs=_row_spec,
        out_shape=jax.ShapeDtypeStruct((NP, D), jnp.float32),
    )(acc_a, acc_b, dinv, b_tiled, w)


def _tc_fin_body(aa_ref, ab_ref, dinv_ref, w_ref, b_ref, o_ref):
    r = (aa_ref[...] + ab_ref[...]) * dinv_ref[...]
    z = jnp.dot(r, w_ref[...], preferred_element_type=jnp.float32) + b_ref[0:1, :]
    m = jnp.max(z, axis=1, keepdims=True)
    zs = z - m
    o_ref[...] = zs - jnp.log(jnp.sum(jnp.exp(zs), axis=1, keepdims=True))


def _tc_fin(acc_a, acc_b, dinv, w2, b2_tiled):
    return pl.pallas_call(
        _tc_fin_body,
        grid=(GRID,),
        in_specs=[
            _row_spec,
            _row_spec,
            _row_spec,
            pl.BlockSpec((D, 2), lambda i: (0, 0)),
            pl.BlockSpec((8, 2), lambda i: (0, 0)),
        ],
        out_specs=pl.BlockSpec((BR, 2), lambda i: (i, 0)),
        out_shape=jax.ShapeDtypeStruct((NP, 2), jnp.float32),
    )(acc_a, acc_b, dinv, w2, b2_tiled)


def kernel(x, edge_index, W1, b1, W3, b3, W2, b2):
    row = edge_index[0].astype(jnp.int32)
    col = edge_index[1].astype(jnp.int32)
    pad = EP - E
    pad_idx = jnp.arange(pad, dtype=jnp.int32)
    row3 = jnp.concatenate(
        [row, pad_idx % N]).reshape(NW, NCHUNK, CHUNK)
    col3 = jnp.concatenate(
        [col, N + pad_idx % (NP - N)]).reshape(NW, NCHUNK, CHUNK)

    x_p = jnp.pad(x, ((0, NP - N), (0, 0)))
    zeros = jnp.zeros((NP, D), jnp.float32)
    ones_chunk = jnp.ones((CHUNK, D), jnp.float32)
    b1_t = jnp.tile(b1.reshape(1, D), (8, 1))
    b3_t = jnp.tile(b3.reshape(1, D), (8, 1))
    b2_t = jnp.tile(b2.reshape(1, 2), (8, 1))
    eye = jnp.eye(D, dtype=jnp.float32)

    deg2 = _sc_deg(col3, ones_chunk, zeros)
    g1, dinv = _tc1(x_p, W1, deg2[0], deg2[1])
    a1 = _sc_agg(g1, row3, col3, zeros)
    g2 = _tc_mid(a1[0], a1[1], dinv, b1_t, W3)
    a2 = _sc_agg(g2, row3, col3, zeros)
    g3 = _tc_mid(a2[0], a2[1], dinv, b3_t, eye)
    a3 = _sc_agg(g3, row3, col3, zeros)
    out = _tc_fin(a3[0], a3[1], dinv, W2, b2_t)
    return out[:N]
